# Initial kernel scaffold; baseline (speedup 1.0000x reference)
import jax
import jax.numpy as jnp
from jax.experimental import pallas as pl


def kernel(x, x_org, adj_values, W_in, b_in, W_res, Wc0, bc0, lnw0, lnb0, Wc1, bc1, lnw1, lnb1, Wc2, bc2, lnw2, lnb2, W_out, b_out, edge_index, adj_indices):
    raise NotImplementedError("write your pallas kernel here")


# speedup vs baseline: 8.7786x; 8.7786x over previous
import functools

import jax
import jax.numpy as jnp
from jax import lax
from jax.experimental import pallas as pl
from jax.experimental.pallas import tpu as pltpu
from jax.experimental.pallas import tpu_sc as plsc

N = 10000
NP = 10240
E = 320000
D = 128

NC = 2
NS = 16
NW = NC * NS
EPW = E // NW
CH = 80
NCHUNK = EPW // CH
RPT = NP // NS
CHR = 80
NQZ = RPT // CHR

_mesh = plsc.VectorSubcoreMesh(core_axis_name="c", subcore_axis_name="s")


@functools.partial(
    pl.kernel,
    out_type=jax.ShapeDtypeStruct((NW, NP), jnp.float32),
    mesh=_mesh,
    scratch_types=[
        pltpu.VMEM((CH,), jnp.int32),
        pltpu.VMEM((NP,), jnp.float32),
    ],
    compiler_params=pltpu.CompilerParams(needs_layout_passes=False),
)
def _deg_pass(col_hbm, degp_hbm, idx_v, deg_v):
    c = lax.axis_index("c")
    s = lax.axis_index("s")
    wid = c * NS + s
    base = wid * EPW
    z16 = jnp.zeros((16,), jnp.float32)
    ones16 = jnp.ones((16,), jnp.float32)
    for i in range(NP // 16):
        deg_v[pl.ds(i * 16, 16)] = z16

    def chunk(k, carry):
        pltpu.sync_copy(col_hbm.at[pl.ds(base + k * CH, CH)], idx_v)
        for g in range(CH // 16):
            ivec = idx_v[pl.ds(g * 16, 16)]
            plsc.addupdate_scatter(deg_v, [ivec], ones16)
        return carry
    lax.fori_loop(0, NCHUNK, chunk, 0)
    pltpu.sync_copy(deg_v, degp_hbm.at[wid])


@functools.partial(
    pl.kernel,
    out_type=jax.ShapeDtypeStruct((NC, NP, D), jnp.float32),
    mesh=_mesh,
    scratch_types=[
        pltpu.VMEM((CH,), jnp.int32),
        pltpu.VMEM((CH,), jnp.int32),
        pltpu.VMEM((NQZ, CHR), jnp.int32),
        pltpu.VMEM((CH, D), jnp.float32),
        pltpu.VMEM((CHR, D), jnp.float32),
        pltpu.VMEM_SHARED((NP, D), jnp.float32),
        pltpu.SemaphoreType.DMA,
    ],
)
def _conv_pass(a_hbm, row_hbm, col_hbm, accp_hbm, ri_v, ci_v, idxz_v, rows_v,
               wb_v, acc_sh, sem):
    c = lax.axis_index("c")
    s = lax.axis_index("s")
    base = (c * NS + s) * EPW
    io = lax.iota(jnp.int32, 16)

    for q in range(NQZ):
        for t in range(CHR // 16):
            idxz_v[q, pl.ds(t * 16, 16)] = s * RPT + q * CHR + t * 16 + io

    def zfill(i, carry):
        for j in range(8):
            wb_v[i, pl.ds(j * 16, 16)] = jnp.zeros((16,), jnp.float32)
        return carry
    lax.fori_loop(0, CHR, zfill, 0)
    for q in range(NQZ):
        pltpu.sync_copy(wb_v, acc_sh.at[idxz_v.at[q]])
    plsc.subcore_barrier()

    def chunk(k, carry):
        pltpu.sync_copy(row_hbm.at[pl.ds(base + k * CH, CH)], ri_v)
        pltpu.sync_copy(col_hbm.at[pl.ds(base + k * CH, CH)], ci_v)
        pltpu.async_copy(a_hbm.at[ri_v], rows_v, sem).wait()
        pltpu.sync_copy(rows_v, acc_sh.at[ci_v], add=True)
        return carry
    lax.fori_loop(0, NCHUNK, chunk, 0)
    plsc.subcore_barrier()

    for q in range(NQZ):
        pltpu.async_copy(acc_sh.at[idxz_v.at[q]], wb_v, sem).wait()
        pltpu.sync_copy(wb_v, accp_hbm.at[c, pl.ds(s * RPT + q * CHR, CHR), :])


@functools.partial(
    pl.kernel,
    out_type=jax.ShapeDtypeStruct((NC, NP, D), jnp.float32),
    mesh=_mesh,
    scratch_types=[
        pltpu.VMEM((CH,), jnp.int32),
        pltpu.VMEM((CH,), jnp.int32),
        pltpu.VMEM((CH,), jnp.float32),
        pltpu.VMEM((NQZ, CHR), jnp.int32),
        pltpu.VMEM((CH, D), jnp.float32),
        pltpu.VMEM((CHR, D), jnp.float32),
        pltpu.VMEM_SHARED((NP, D), jnp.float32),
        pltpu.SemaphoreType.DMA,
    ],
)
def _res_pass(sp_hbm, src_hbm, dst_hbm, val_hbm, resp_hbm, ri_v, ci_v, vv_v,
              idxz_v, rows_v, wb_v, acc_sh, sem):
    c = lax.axis_index("c")
    s = lax.axis_index("s")
    base = (c * NS + s) * EPW
    io = lax.iota(jnp.int32, 16)

    for q in range(NQZ):
        for t in range(CHR // 16):
            idxz_v[q, pl.ds(t * 16, 16)] = s * RPT + q * CHR + t * 16 + io

    def zfill(i, carry):
        for j in range(8):
            wb_v[i, pl.ds(j * 16, 16)] = jnp.zeros((16,), jnp.float32)
        return carry
    lax.fori_loop(0, CHR, zfill, 0)
    for q in range(NQZ):
        pltpu.sync_copy(wb_v, acc_sh.at[idxz_v.at[q]])
    plsc.subcore_barrier()

    def chunk(k, carry):
        pltpu.sync_copy(src_hbm.at[pl.ds(base + k * CH, CH)], ri_v)
        pltpu.sync_copy(dst_hbm.at[pl.ds(base + k * CH, CH)], ci_v)
        pltpu.sync_copy(val_hbm.at[pl.ds(base + k * CH, CH)], vv_v)
        pltpu.async_copy(sp_hbm.at[ri_v], rows_v, sem).wait()

        for g in range(CH // 16):
            vvec = vv_v[pl.ds(g * 16, 16)]
            for t in range(16):
                v = vvec[t]
                e = g * 16 + t
                for j in range(8):
                    sl = pl.ds(j * 16, 16)
                    rows_v[e, sl] = rows_v[e, sl] * v
        pltpu.sync_copy(rows_v, acc_sh.at[ci_v], add=True)
        return carry
    lax.fori_loop(0, NCHUNK, chunk, 0)
    plsc.subcore_barrier()

    for q in range(NQZ):
        pltpu.async_copy(acc_sh.at[idxz_v.at[q]], wb_v, sem).wait()
        pltpu.sync_copy(wb_v, resp_hbm.at[c, pl.ds(s * RPT + q * CHR, CHR), :])


def _pre_body(x_ref, xo_ref, wi_ref, bi_ref, wr_ref, h0_ref, sp_ref):
    h0 = jnp.dot(x_ref[...], wi_ref[...], preferred_element_type=jnp.float32)
    h0_ref[...] = jnp.maximum(h0 + bi_ref[...], 0.0)
    sp_ref[...] = jnp.dot(xo_ref[...], wr_ref[...],
                          preferred_element_type=jnp.float32)


def _dinv_a0_body(degp_ref, h0_ref, wc0_ref, dinv_ref, a0_ref):
    d = jnp.sum(degp_ref[...], axis=0)[:N, None] + 1.0
    dinv = jnp.broadcast_to(lax.rsqrt(d), (N, D))
    dinv_ref[...] = dinv
    a0_ref[...] = dinv * jnp.dot(h0_ref[...], wc0_ref[...],
                                 preferred_element_type=jnp.float32)


def _ln_relu(t, lnw, lnb):
    mu = jnp.mean(t, axis=-1, keepdims=True)
    var = jnp.mean((t - mu) ** 2, axis=-1, keepdims=True)
    return jnp.maximum((t - mu) / jnp.sqrt(var + 1e-5) * lnw + lnb, 0.0)


def _comb_body(accp_ref, a_ref, dinv_ref, bc_ref, lnw_ref, lnb_ref, wcn_ref,
               an_ref):
    dinv = dinv_ref[...]
    t = dinv * (accp_ref[0, :N] + accp_ref[1, :N] + a_ref[...]) + bc_ref[...]
    hn = _ln_relu(t, lnw_ref[...], lnb_ref[...])
    an_ref[...] = dinv * jnp.dot(hn, wcn_ref[...],
                                 preferred_element_type=jnp.float32)


def _final_body(accp_ref, a_ref, dinv_ref, bc_ref, lnw_ref, lnb_ref, wo_ref,
                bo_ref, resp_ref, out_ref, res_ref):
    t = dinv_ref[...] * (accp_ref[0, :N] + accp_ref[1, :N] + a_ref[...]) + bc_ref[...]
    hn = _ln_relu(t, lnw_ref[...], lnb_ref[...])
    out_ref[...] = jnp.dot(hn, wo_ref[...],
                           preferred_element_type=jnp.float32) + bo_ref[...]
    res_ref[...] = resp_ref[0, :N] + resp_ref[1, :N]


_f32 = jnp.float32
_nd = jax.ShapeDtypeStruct((N, D), _f32)

_pre = pl.pallas_call(_pre_body, out_shape=(_nd, _nd))
_dinv_a0 = pl.pallas_call(_dinv_a0_body, out_shape=(_nd, _nd))
_comb = pl.pallas_call(_comb_body, out_shape=_nd)
_final = pl.pallas_call(_final_body, out_shape=(_nd, _nd))


@jax.jit
def kernel(x, x_org, adj_values, W_in, b_in, W_res, Wc0, bc0, lnw0, lnb0,
           Wc1, bc1, lnw1, lnb1, Wc2, bc2, lnw2, lnb2, W_out, b_out,
           edge_index, adj_indices):
    row = edge_index[0]
    col = edge_index[1]
    dst = adj_indices[0]
    src = adj_indices[1]

    degp = _deg_pass(col)
    h0, sp = _pre(x, x_org, W_in, b_in, W_res)
    resp = _res_pass(sp, src, dst, adj_values)
    dinv, a0 = _dinv_a0(degp, h0, Wc0)
    accp = _conv_pass(a0, row, col)
    a1 = _comb(accp, a0, dinv, bc0, lnw0, lnb0, Wc1)
    accp = _conv_pass(a1, row, col)
    a2 = _comb(accp, a1, dinv, bc1, lnw1, lnb1, Wc2)
    accp = _conv_pass(a2, row, col)
    out, residual = _final(accp, a2, dinv, bc2, lnw2, lnb2, W_out, b_out, resp)
    return out, residual
